# Initial kernel scaffold; baseline (speedup 1.0000x reference)
import jax
import jax.numpy as jnp
from jax.experimental import pallas as pl


def kernel(input_ids, embedding_matrix):
    raise NotImplementedError("write your pallas kernel here")


# speedup vs baseline: 1.1036x; 1.1036x over previous
import jax
import jax.numpy as jnp
from jax import lax
from jax.experimental import pallas as pl
from jax.experimental.pallas import tpu as pltpu
from jax.experimental.pallas import tpu_sc as plsc

_info = plsc.get_sparse_core_info()
_NC = _info.num_cores
_NS = _info.num_subcores
_NW = _NC * _NS

_CHUNK = 1600


def _emb_body(idx_hbm, tab_hbm, out_hbm, idx_v, rows_v, sem):
    total = idx_hbm.shape[0]
    per_w = total // _NW
    nchunk = per_w // _CHUNK
    wid = lax.axis_index("s") * _NC + lax.axis_index("c")
    base0 = wid * per_w

    def body(j, carry):
        base = base0 + j * _CHUNK
        pltpu.sync_copy(idx_hbm.at[pl.ds(base, _CHUNK)], idx_v)
        pltpu.async_copy(tab_hbm.at[idx_v], rows_v, sem).wait()
        pltpu.sync_copy(rows_v, out_hbm.at[pl.ds(base, _CHUNK)])
        return carry

    lax.fori_loop(0, nchunk, body, 0)


def _emb(idx_flat, table):
    total = idx_flat.shape[0]
    dim = table.shape[1]
    return pl.kernel(
        _emb_body,
        out_type=jax.ShapeDtypeStruct((total, dim), table.dtype),
        mesh=plsc.VectorSubcoreMesh(core_axis_name="c", subcore_axis_name="s"),
        scratch_types=[
            pltpu.VMEM((_CHUNK,), jnp.int32),
            pltpu.VMEM((_CHUNK, dim), jnp.float32),
            pltpu.SemaphoreType.DMA,
        ],
        compiler_params=pltpu.CompilerParams(use_tc_tiling_on_sc=False),
    )(idx_flat, table)


def kernel(input_ids, embedding_matrix):
    b, h = input_ids.shape
    idx_flat = input_ids.reshape(-1).astype(jnp.int32)
    out = _emb(idx_flat, embedding_matrix)
    return out.reshape(b, h, embedding_matrix.shape[1])


# speedup vs baseline: 1.1134x; 1.0089x over previous
import jax
import jax.numpy as jnp
from jax import lax
from jax.experimental import pallas as pl
from jax.experimental.pallas import tpu as pltpu
from jax.experimental.pallas import tpu_sc as plsc

_info = plsc.get_sparse_core_info()
_NC = _info.num_cores
_NS = _info.num_subcores
_NW = _NC * _NS

_CHUNK = 1280
_NBUF = 2


def _emb_body(idx_hbm, tab_hbm, out_hbm, idx_v, rows_v, sem_g, sem_w):
    total = idx_hbm.shape[0]
    per_w = total // _NW
    nchunk = per_w // _CHUNK
    wid = lax.axis_index("s") * _NC + lax.axis_index("c")
    base0 = wid * per_w

    pltpu.sync_copy(idx_hbm.at[pl.ds(base0, per_w)], idx_v)

    def start_gather(j):
        b = j % _NBUF
        return pltpu.async_copy(
            tab_hbm.at[idx_v.at[pl.ds(j * _CHUNK, _CHUNK)]],
            rows_v.at[b], sem_g.at[b])

    def start_write(j):
        b = j % _NBUF
        return pltpu.async_copy(
            rows_v.at[b],
            out_hbm.at[pl.ds(base0 + j * _CHUNK, _CHUNK)], sem_w.at[b])

    writes = [None] * _NBUF
    gathers = [None] * _NBUF
    gathers[0] = start_gather(0)
    for j in range(nchunk):
        b = j % _NBUF
        if j + 1 < nchunk:
            b2 = (j + 1) % _NBUF
            if writes[b2] is not None:
                writes[b2].wait()
            gathers[b2] = start_gather(j + 1)
        gathers[b].wait()
        writes[b] = start_write(j)
    for w in writes:
        if w is not None:
            w.wait()


def _emb(idx_flat, table):
    total = idx_flat.shape[0]
    dim = table.shape[1]
    per_w = total // _NW
    return pl.kernel(
        _emb_body,
        out_type=jax.ShapeDtypeStruct((total, dim), table.dtype),
        mesh=plsc.VectorSubcoreMesh(core_axis_name="c", subcore_axis_name="s"),
        scratch_types=[
            pltpu.VMEM((per_w,), jnp.int32),
            pltpu.VMEM((_NBUF, _CHUNK, dim), jnp.float32),
            pltpu.SemaphoreType.DMA((_NBUF,)),
            pltpu.SemaphoreType.DMA((_NBUF,)),
        ],
        compiler_params=pltpu.CompilerParams(use_tc_tiling_on_sc=False),
    )(idx_flat, table)


def kernel(input_ids, embedding_matrix):
    b, h = input_ids.shape
    idx_flat = input_ids.reshape(-1).astype(jnp.int32)
    out = _emb(idx_flat, embedding_matrix)
    return out.reshape(b, h, embedding_matrix.shape[1])


# speedup vs baseline: 1.1141x; 1.0006x over previous
import jax
import jax.numpy as jnp
from jax import lax
from jax.experimental import pallas as pl
from jax.experimental.pallas import tpu as pltpu
from jax.experimental.pallas import tpu_sc as plsc

_info = plsc.get_sparse_core_info()
_NC = _info.num_cores
_NS = _info.num_subcores
_NW = _NC * _NS

_CHUNK = 320
_R = 8
_K = 4


def _emb_body(idx_hbm, tab_hbm, out_hbm, idx_v, rows_v, sem_g, sem_w):
    total = idx_hbm.shape[0]
    per_w = total // _NW
    nchunk = per_w // _CHUNK
    ngroups = nchunk // _R
    wid = lax.axis_index("s") * _NC + lax.axis_index("c")
    base0 = wid * per_w

    pltpu.sync_copy(idx_hbm.at[pl.ds(base0, per_w)], idx_v)

    def gather_start(j, b):
        pltpu.async_copy(
            tab_hbm.at[idx_v.at[pl.ds(j * _CHUNK, _CHUNK)]],
            rows_v.at[b], sem_g.at[b])

    def gather_wait(b):
        pltpu.make_async_copy(
            out_hbm.at[pl.ds(0, _CHUNK)], rows_v.at[b], sem_g.at[b]).wait()

    def write_start(j, b):
        pltpu.async_copy(
            rows_v.at[b],
            out_hbm.at[pl.ds(base0 + j * _CHUNK, _CHUNK)], sem_w.at[b])

    def write_wait(b):
        pltpu.make_async_copy(
            rows_v.at[b], out_hbm.at[pl.ds(0, _CHUNK)], sem_w.at[b]).wait()

    for j in range(_K):
        gather_start(j, j)
    for j in range(_R):
        b = j % _R
        gather_wait(b)
        write_start(j, b)
        bn = (j + _K) % _R
        if j >= _R - _K:
            write_wait(bn)
        gather_start(j + _K, bn)

    def group_body(g, carry):
        j0 = g * _R
        for i in range(_R):
            j = j0 + i
            b = i
            gather_wait(b)
            write_start(j, b)
            bn = (i + _K) % _R
            write_wait(bn)
            gather_start(j + _K, bn)
        return carry

    lax.fori_loop(1, ngroups - 1, group_body, 0)

    for i in range(_R):
        j = nchunk - _R + i
        b = i
        gather_wait(b)
        write_start(j, b)
        if i < _R - _K:
            bn = (i + _K) % _R
            write_wait(bn)
            gather_start(j + _K, bn)
    for b in range(_R):
        write_wait(b)


def _emb(idx_flat, table):
    total = idx_flat.shape[0]
    dim = table.shape[1]
    per_w = total // _NW
    return pl.kernel(
        _emb_body,
        out_type=jax.ShapeDtypeStruct((total, dim), table.dtype),
        mesh=plsc.VectorSubcoreMesh(core_axis_name="c", subcore_axis_name="s"),
        scratch_types=[
            pltpu.VMEM((per_w,), jnp.int32),
            pltpu.VMEM((_R, _CHUNK, dim), jnp.float32),
            pltpu.SemaphoreType.DMA((_R,)),
            pltpu.SemaphoreType.DMA((_R,)),
        ],
        compiler_params=pltpu.CompilerParams(use_tc_tiling_on_sc=False),
    )(idx_flat, table)


def kernel(input_ids, embedding_matrix):
    b, h = input_ids.shape
    idx_flat = input_ids.reshape(-1).astype(jnp.int32)
    out = _emb(idx_flat, embedding_matrix)
    return out.reshape(b, h, embedding_matrix.shape[1])


# speedup vs baseline: 1.6355x; 1.4681x over previous
import jax
import jax.numpy as jnp
from jax import lax
from jax.experimental import pallas as pl
from jax.experimental.pallas import tpu as pltpu
from jax.experimental.pallas import tpu_sc as plsc

_info = plsc.get_sparse_core_info()
_NC = _info.num_cores
_NS = _info.num_subcores
_NW = _NC * _NS

_H = 50
_B = 16384
_D = 32
_BW = _B // _NW
_TB = _BW // 128
_NU = _H * _TB


def _emb_body(ids_t, tpack, out_t, ids_buf, q_buf, r_buf, fet, out_loc,
              sem_g, sem_w):
    wid = lax.axis_index("s") * _NC + lax.axis_index("c")
    b0 = wid * _BW

    pltpu.sync_copy(ids_t.at[:, pl.ds(b0, _BW)], ids_buf)

    iota = lax.iota(jnp.int32, 16)

    def qr(u, slot):
        h = lax.shift_right_logical(u, 2)
        tb = lax.bitwise_and(u, 3)
        for v in range(8):
            idx = ids_buf[h, pl.ds(tb * 128 + 16 * v, 16)]
            q_buf[slot, pl.ds(16 * v, 16)] = lax.shift_right_logical(idx, 2)
            r_buf[slot, pl.ds(16 * v, 16)] = lax.bitwise_and(idx, 3)

    def gather_start(slot):
        pltpu.async_copy(tpack.at[q_buf.at[slot]], fet.at[slot],
                         sem_g.at[slot])

    def gather_wait(slot):
        pltpu.make_async_copy(tpack.at[pl.ds(0, 128)], fet.at[slot],
                              sem_g.at[slot]).wait()

    def write_start(u, slot):
        h = lax.shift_right_logical(u, 2)
        tb = lax.bitwise_and(u, 3)
        pltpu.async_copy(out_loc.at[slot],
                         out_t.at[h, :, pl.ds(b0 + tb * 128, 128)],
                         sem_w.at[slot])

    def write_wait(slot):
        pltpu.make_async_copy(out_loc.at[slot],
                              out_t.at[0, :, pl.ds(0, 128)],
                              sem_w.at[slot]).wait()

    def extract(slot):
        rv = [r_buf[slot, pl.ds(16 * v, 16)] * 32 for v in range(8)]
        rows = [iota + (16 * v) for v in range(8)]
        for j in range(_D):
            for v in range(8):
                vals = plsc.load_gather(fet.at[slot], [rows[v], rv[v] + j])
                out_loc[slot, j, pl.ds(16 * v, 16)] = vals

    qr(0, 0)
    gather_start(0)

    def body(u, carry):
        slot = lax.bitwise_and(u, 1)
        nslot = lax.bitwise_and(u + 1, 1)

        @pl.when(u + 1 < _NU)
        def _():
            qr(u + 1, nslot)
            gather_start(nslot)

        gather_wait(slot)

        @pl.when(u >= 2)
        def _():
            write_wait(slot)

        extract(slot)
        write_start(u, slot)
        return carry

    lax.fori_loop(0, _NU, body, 0)
    write_wait(0)
    write_wait(1)


def _emb(ids_t, tpack):
    return pl.kernel(
        _emb_body,
        out_type=jax.ShapeDtypeStruct((_H, _D, _B), jnp.float32),
        mesh=plsc.VectorSubcoreMesh(core_axis_name="c", subcore_axis_name="s"),
        scratch_types=[
            pltpu.VMEM((_H, _BW), jnp.int32),
            pltpu.VMEM((2, 128), jnp.int32),
            pltpu.VMEM((2, 128), jnp.int32),
            pltpu.VMEM((2, 128, 128), jnp.float32),
            pltpu.VMEM((2, _D, 128), jnp.float32),
            pltpu.SemaphoreType.DMA((2,)),
            pltpu.SemaphoreType.DMA((2,)),
        ],
        compiler_params=pltpu.CompilerParams(needs_layout_passes=False),
    )(ids_t, tpack)


def kernel(input_ids, embedding_matrix):
    ids_t = input_ids.T
    tpack = embedding_matrix.reshape(-1, 128)
    out_t = _emb(ids_t.astype(jnp.int32), tpack)
    return out_t.transpose(2, 0, 1)
